# Initial kernel scaffold; baseline (speedup 1.0000x reference)
import jax
import jax.numpy as jnp
from jax.experimental import pallas as pl


def kernel(x, edge_index, drug_index, label, W1, b1, ge1, lge1, W2, b2, ge2, lge2, W3, b3, ge3, lge3, P1, P2):
    raise NotImplementedError("write your pallas kernel here")


# speedup vs baseline: 3.8979x; 3.8979x over previous
import dataclasses
import functools

import jax
import jax.numpy as jnp
from jax import lax
from jax.experimental import pallas as pl
from jax.experimental.pallas import tpu as pltpu
from jax.experimental.pallas import tpu_sc as plsc

NODE_NUM = 8040
GRAPH_BATCH = 8
N = NODE_NUM * GRAPH_BATCH
E = 125000 * GRAPH_BATCH
NUM_DRUG_EDGE = 25000
NUM_DRUG = 38
F = 64
FQ = 16

NCORE = 2
NSUB = 16
CH = 128

EPS = 62720
E_PAD = NSUB * EPS
PAD = E_PAD - E
EPW_DEG = E_PAD // (NCORE * NSUB)

PART = 4024
LAST = N - 15 * PART

CPY = 96
NCPY = N // CPY

BLK = 6432
HIGH = lax.Precision.HIGHEST


@functools.lru_cache(maxsize=None)
def _sc_params():
    cp = pltpu.CompilerParams()
    if "needs_layout_passes" in pltpu.CompilerParams.__dataclass_fields__:
        cp = dataclasses.replace(cp, needs_layout_passes=False)
    if "use_tc_tiling_on_sc" in pltpu.CompilerParams.__dataclass_fields__:
        cp = dataclasses.replace(cp, use_tc_tiling_on_sc=False)
    return cp


@functools.lru_cache(maxsize=None)
def _mesh():
    return plsc.VectorSubcoreMesh(core_axis_name="c", subcore_axis_name="s",
                                  num_cores=NCORE, num_subcores=NSUB)


def _sc_degree(colp, z16, ones16):
    @functools.partial(
        pl.kernel,
        out_type=jax.ShapeDtypeStruct((NCORE, N, 16), jnp.float32),
        mesh=_mesh(),
        scratch_types=[
            pltpu.VMEM_SHARED((N, 16), jnp.float32),
            pltpu.VMEM((1, CH), jnp.int32),
            pltpu.VMEM((CH, 16), jnp.float32),
        ],
        compiler_params=_sc_params(),
    )
    def deg_kernel(col_hbm, z_hbm, ones_hbm, out_hbm, acc_s, ci, ones_v):
        c = lax.axis_index("c")
        s = lax.axis_index("s")
        wid = s * NCORE + c
        off = s * PART

        @pl.when(s < 15)
        def _():
            pltpu.sync_copy(z_hbm, acc_s.at[pl.ds(off, PART)])

        @pl.when(s == 15)
        def _():
            pltpu.sync_copy(z_hbm.at[pl.ds(0, LAST)],
                            acc_s.at[pl.ds(off, LAST)])

        pltpu.sync_copy(ones_hbm, ones_v)
        plsc.subcore_barrier()

        @pl.loop(0, EPW_DEG // CH)
        def _(ch):
            base = wid * EPW_DEG + ch * CH
            pltpu.sync_copy(col_hbm.at[pl.ds(base, CH)], ci.at[0])
            pltpu.sync_copy(ones_v, acc_s.at[ci.at[0]], add=True)

        plsc.subcore_barrier()

        @pl.when(s < 15)
        def _():
            pltpu.sync_copy(acc_s.at[pl.ds(off, PART)],
                            out_hbm.at[c, pl.ds(off, PART)])

        @pl.when(s == 15)
        def _():
            pltpu.sync_copy(acc_s.at[pl.ds(off, LAST)],
                            out_hbm.at[c, pl.ds(off, LAST)])

    return deg_kernel(colp, z16, ones16)


_GATHER_DNUMS = lax.GatherDimensionNumbers(
    offset_dims=(), collapsed_slice_dims=(0,), start_index_map=(0,))


def _lane_splat(vec16, j):
    idx = jnp.full((16, 1), j, jnp.int32)
    return lax.gather(vec16, idx, _GATHER_DNUMS, (1,),
                      mode=lax.GatherScatterMode.PROMISE_IN_BOUNDS)


def _sc_message_pass(h4, rowp, colp, wp, z16):
    @functools.partial(
        pl.kernel,
        out_type=jax.ShapeDtypeStruct((4 * N, FQ), jnp.float32),
        mesh=_mesh(),
        scratch_types=[
            pltpu.VMEM_SHARED((N, FQ), jnp.float32),
            pltpu.VMEM((1, CH), jnp.int32),
            pltpu.VMEM((1, CH), jnp.int32),
            pltpu.VMEM((CH,), jnp.float32),
            pltpu.VMEM((CH, FQ), jnp.float32),
            pltpu.VMEM((1, CPY), jnp.int32),
            pltpu.VMEM((CPY, FQ), jnp.float32),
        ],
        compiler_params=_sc_params(),
    )
    def mp_kernel(h_hbm, row_hbm, col_hbm, w_hbm, z_hbm, out_hbm,
                  acc_s, ri, ci, wv, gb, oi, cb):
        c = lax.axis_index("c")
        s = lax.axis_index("s")
        off = s * PART
        iota4 = lax.iota(jnp.int32, 16) * 4

        for p in range(2):
            q = 2 * p + c

            @pl.when(s < 15)
            def _():
                pltpu.sync_copy(z_hbm, acc_s.at[pl.ds(off, PART)])

            @pl.when(s == 15)
            def _():
                pltpu.sync_copy(z_hbm.at[pl.ds(0, LAST)],
                                acc_s.at[pl.ds(off, LAST)])

            plsc.subcore_barrier()

            @pl.loop(0, EPS // CH)
            def _(ch):
                base = s * EPS + ch * CH
                pltpu.sync_copy(row_hbm.at[pl.ds(base, CH)], ri.at[0])
                pltpu.sync_copy(col_hbm.at[pl.ds(base, CH)], ci.at[0])
                pltpu.sync_copy(w_hbm.at[pl.ds(base, CH)], wv)
                for k in range(CH // 16):
                    sl = pl.ds(k * 16, 16)
                    ri[0, sl] = ri[0, sl] * 4 + q
                pltpu.sync_copy(h_hbm.at[ri.at[0]], gb)

                for g in range(CH // 16):
                    w16 = wv[pl.ds(g * 16, 16)]
                    for j in range(16):
                        e = g * 16 + j
                        w = _lane_splat(w16, j)
                        gb[e, pl.ds(0, FQ)] = gb[e, pl.ds(0, FQ)] * w

                pltpu.sync_copy(gb, acc_s.at[ci.at[0]], add=True)

            plsc.subcore_barrier()

            @pl.loop(0, (NCPY + NSUB - 1) // NSUB)
            def _(k):
                t = s + k * NSUB

                @pl.when(t < NCPY)
                def _():
                    for g in range(CPY // 16):
                        oi[0, pl.ds(g * 16, 16)] = (
                            iota4 + ((t * CPY + g * 16) * 4 + q))
                    pltpu.sync_copy(acc_s.at[pl.ds(t * CPY, CPY)], cb)
                    pltpu.sync_copy(cb, out_hbm.at[oi.at[0]])

            if p == 0:
                plsc.subcore_barrier()

    return mp_kernel(h4, rowp, colp, wp, z16)


def _mm1_body(x_ref, w_ref, b_ref, o_ref):
    o_ref[...] = jnp.dot(x_ref[...], w_ref[...], precision=HIGH) + b_ref[...]


def _tc_mm1(x, W1, b1):
    return pl.pallas_call(
        _mm1_body,
        grid=(N // BLK,),
        in_specs=[
            pl.BlockSpec((BLK, F), lambda i: (i, 0)),
            pl.BlockSpec((F, F), lambda i: (0, 0)),
            pl.BlockSpec((1, F), lambda i: (0, 0)),
        ],
        out_specs=pl.BlockSpec((BLK, F), lambda i: (i, 0)),
        out_shape=jax.ShapeDtypeStruct((N, F), jnp.float32),
    )(x, W1, b1.reshape(1, F))


def _ds_body(degp_ref, t_ref, ds_ref, hs_ref):
    i = pl.program_id(0)
    deg = degp_ref[0][:, 0:1] + degp_ref[1][:, 0:1] + 1.0
    rows = lax.broadcasted_iota(jnp.int32, deg.shape, 0)
    deg = jnp.where((rows == 0) & (i == 0), deg - float(PAD), deg)
    ds = jnp.broadcast_to(lax.rsqrt(deg), t_ref.shape)
    ds_ref[...] = ds
    hs_ref[...] = ds * t_ref[...]


def _tc_ds_hs(degp, t1):
    blk = 3216
    return pl.pallas_call(
        _ds_body,
        grid=(N // blk,),
        in_specs=[
            pl.BlockSpec((2, blk, 16), lambda i: (0, i, 0)),
            pl.BlockSpec((blk, F), lambda i: (i, 0)),
        ],
        out_specs=[
            pl.BlockSpec((blk, F), lambda i: (i, 0)),
            pl.BlockSpec((blk, F), lambda i: (i, 0)),
        ],
        out_shape=[
            jax.ShapeDtypeStruct((N, F), jnp.float32),
            jax.ShapeDtypeStruct((N, F), jnp.float32),
        ],
    )(degp, t1)


def _layer_body(acc_ref, hs_ref, ds_ref, lw_ref, w_ref, b_ref, o_ref):
    ds = ds_ref[...]
    act = jax.nn.relu(ds * (acc_ref[...] + lw_ref[...] * hs_ref[...]))
    o_ref[...] = ds * (jnp.dot(act, w_ref[...], precision=HIGH) + b_ref[...])


def _tc_layer(acc, hs, ds, lw, W, b):
    return pl.pallas_call(
        _layer_body,
        grid=(N // BLK,),
        in_specs=[
            pl.BlockSpec((BLK, F), lambda i: (i, 0)),
            pl.BlockSpec((BLK, F), lambda i: (i, 0)),
            pl.BlockSpec((BLK, F), lambda i: (i, 0)),
            pl.BlockSpec((BLK, F), lambda i: (i, 0)),
            pl.BlockSpec((F, F), lambda i: (0, 0)),
            pl.BlockSpec((1, F), lambda i: (0, 0)),
        ],
        out_specs=pl.BlockSpec((BLK, F), lambda i: (i, 0)),
        out_shape=jax.ShapeDtypeStruct((N, F), jnp.float32),
    )(acc, hs, ds, lw, W, b.reshape(1, F))


def _final_body(acc_ref, hs_ref, ds_ref, lw_ref, o_ref):
    o_ref[...] = jax.nn.relu(
        ds_ref[...] * (acc_ref[...] + lw_ref[...] * hs_ref[...]))


def _tc_final(acc, hs, ds, lw):
    return pl.pallas_call(
        _final_body,
        grid=(N // BLK,),
        in_specs=[
            pl.BlockSpec((BLK, F), lambda i: (i, 0)),
            pl.BlockSpec((BLK, F), lambda i: (i, 0)),
            pl.BlockSpec((BLK, F), lambda i: (i, 0)),
            pl.BlockSpec((BLK, F), lambda i: (i, 0)),
        ],
        out_specs=pl.BlockSpec((BLK, F), lambda i: (i, 0)),
        out_shape=jax.ShapeDtypeStruct((N, F), jnp.float32),
    )(acc, hs, ds, lw)


def _decoder_body(h_ref, ai_ref, bi_ref, p1_ref, p2_ref, o_ref, a_scr, b_scr):
    def gather(i, _):
        a_scr[pl.ds(i, 1)] = h_ref[pl.ds(ai_ref[i], 1)]
        b_scr[pl.ds(i, 1)] = h_ref[pl.ds(bi_ref[i], 1)]
        return 0

    lax.fori_loop(0, 512, gather, 0)
    p1 = p1_ref[...]
    q = jnp.dot(jnp.dot(p1, p2_ref[...], precision=HIGH), p1.T,
                precision=HIGH)
    p = jnp.dot(a_scr[...], q, precision=HIGH)
    o_ref[...] = jnp.sum(p * b_scr[...], axis=1, keepdims=True)


def _tc_decoder(h3, ai, bi, P1, P2):
    return pl.pallas_call(
        _decoder_body,
        in_specs=[
            pl.BlockSpec(memory_space=pltpu.VMEM),
            pl.BlockSpec(memory_space=pltpu.SMEM),
            pl.BlockSpec(memory_space=pltpu.SMEM),
            pl.BlockSpec(memory_space=pltpu.VMEM),
            pl.BlockSpec(memory_space=pltpu.VMEM),
        ],
        out_specs=pl.BlockSpec(memory_space=pltpu.VMEM),
        out_shape=jax.ShapeDtypeStruct((512, 1), jnp.float32),
        scratch_shapes=[
            pltpu.VMEM((512, F), jnp.float32),
            pltpu.VMEM((512, F), jnp.float32),
        ],
    )(h3, ai, bi, P1, P2)


def kernel(x, edge_index, drug_index, label, W1, b1, ge1, lge1, W2, b2, ge2,
           lge2, W3, b3, ge3, lge3, P1, P2):
    del label
    i32 = jnp.int32
    f32 = jnp.float32

    row = edge_index[0].astype(i32)
    col = edge_index[1].astype(i32)
    zpad = jnp.zeros((PAD,), i32)
    rowp = jnp.concatenate([row, zpad])
    colp = jnp.concatenate([col, zpad])

    ones_drug = jnp.ones((NUM_DRUG_EDGE,), f32)
    wpad = jnp.zeros((PAD,), f32)

    def edge_w(ge):
        return jnp.concatenate(
            [jnp.tile(jnp.concatenate([ge, ones_drug]), GRAPH_BATCH), wpad])

    ones_loop = jnp.ones((NUM_DRUG,), f32)

    def loop_w(lge):
        lw = jnp.tile(jnp.concatenate([lge, ones_loop]), GRAPH_BATCH)
        return jnp.broadcast_to(lw[:, None], (N, F))

    z16 = jnp.zeros((PART, 16), f32)
    ones16 = jnp.ones((CH, 16), f32)

    degp = _sc_degree(colp, z16, ones16)
    t1 = _tc_mm1(x, W1, b1)
    ds, hs = _tc_ds_hs(degp, t1)

    acc = _sc_message_pass(hs.reshape(4 * N, FQ), rowp, colp,
                           edge_w(ge1), z16).reshape(N, F)
    hs = _tc_layer(acc, hs, ds, loop_w(lge1), W2, b2)
    acc = _sc_message_pass(hs.reshape(4 * N, FQ), rowp, colp,
                           edge_w(ge2), z16).reshape(N, F)
    hs = _tc_layer(acc, hs, ds, loop_w(lge2), W3, b3)
    acc = _sc_message_pass(hs.reshape(4 * N, FQ), rowp, colp,
                           edge_w(ge3), z16).reshape(N, F)
    h3 = _tc_final(acc, hs, ds, loop_w(lge3))

    idx = drug_index.reshape(-1, 2).astype(i32)
    ai = (idx[:, 0] - 1) % N
    bi = (idx[:, 1] - 1) % N
    return _tc_decoder(h3, ai, bi, P1, P2)


# speedup vs baseline: 11.1023x; 2.8483x over previous
import dataclasses
import functools

import jax
import jax.numpy as jnp
from jax import lax
from jax.experimental import pallas as pl
from jax.experimental.pallas import tpu as pltpu
from jax.experimental.pallas import tpu_sc as plsc

NODE_NUM = 8040
GRAPH_BATCH = 8
N = NODE_NUM * GRAPH_BATCH
E = 125000 * GRAPH_BATCH
NUM_DRUG_EDGE = 25000
NUM_DRUG = 38
F = 64
FQ = 16

NCORE = 2
NSUB = 16
CH = 128
SUP = 7

EPS = 62720
E_PAD = NSUB * EPS
PAD = E_PAD - E
EPW_DEG = E_PAD // (NCORE * NSUB)

PART = 4024
LAST = N - 15 * PART

CPY = 96
NCPY = N // CPY

BLK = 6432
HIGH = lax.Precision.HIGHEST


@functools.lru_cache(maxsize=None)
def _sc_params():
    cp = pltpu.CompilerParams()
    if "needs_layout_passes" in pltpu.CompilerParams.__dataclass_fields__:
        cp = dataclasses.replace(cp, needs_layout_passes=False)
    if "use_tc_tiling_on_sc" in pltpu.CompilerParams.__dataclass_fields__:
        cp = dataclasses.replace(cp, use_tc_tiling_on_sc=False)
    return cp


@functools.lru_cache(maxsize=None)
def _mesh():
    return plsc.VectorSubcoreMesh(core_axis_name="c", subcore_axis_name="s",
                                  num_cores=NCORE, num_subcores=NSUB)


def _sc_degree(col2, z16, ones16):
    rows_w = EPW_DEG // CH

    @functools.partial(
        pl.kernel,
        out_type=jax.ShapeDtypeStruct((NCORE, N, 16), jnp.float32),
        mesh=_mesh(),
        scratch_types=[
            pltpu.VMEM_SHARED((N, 16), jnp.float32),
            pltpu.VMEM((SUP, CH), jnp.int32),
            pltpu.VMEM((CH, 16), jnp.float32),
            pltpu.SemaphoreType.DMA,
        ],
        compiler_params=_sc_params(),
    )
    def deg_kernel(col_hbm, z_hbm, ones_hbm, out_hbm, acc_s, ci, ones_v, sem):
        c = lax.axis_index("c")
        s = lax.axis_index("s")
        wid = s * NCORE + c
        off = s * PART

        @pl.when(s < 15)
        def _():
            pltpu.sync_copy(z_hbm, acc_s.at[pl.ds(off, PART)])

        @pl.when(s == 15)
        def _():
            pltpu.sync_copy(z_hbm.at[pl.ds(0, LAST)],
                            acc_s.at[pl.ds(off, LAST)])

        pltpu.sync_copy(ones_hbm, ones_v)
        plsc.subcore_barrier()

        @pl.loop(0, rows_w // SUP)
        def _(u):
            base = wid * rows_w + u * SUP
            pltpu.sync_copy(col_hbm.at[pl.ds(base, SUP)], ci)
            ds_ = [pltpu.async_copy(ones_v, acc_s.at[ci.at[b]], sem,
                                    add=True) for b in range(SUP)]
            for d in ds_:
                d.wait()

        plsc.subcore_barrier()

        @pl.when(s < 15)
        def _():
            pltpu.sync_copy(acc_s.at[pl.ds(off, PART)],
                            out_hbm.at[c, pl.ds(off, PART)])

        @pl.when(s == 15)
        def _():
            pltpu.sync_copy(acc_s.at[pl.ds(off, LAST)],
                            out_hbm.at[c, pl.ds(off, LAST)])

    return deg_kernel(col2, z16, ones16)


_GATHER_DNUMS = lax.GatherDimensionNumbers(
    offset_dims=(), collapsed_slice_dims=(0,), start_index_map=(0,))


def _lane_splat(vec16, j):
    idx = jnp.full((16, 1), j, jnp.int32)
    return lax.gather(vec16, idx, _GATHER_DNUMS, (1,),
                      mode=lax.GatherScatterMode.PROMISE_IN_BOUNDS)


def _sc_message_pass(h4, row2, col2, w2, z16):
    rows_s = EPS // CH

    @functools.partial(
        pl.kernel,
        out_type=jax.ShapeDtypeStruct((4 * N, FQ), jnp.float32),
        mesh=_mesh(),
        scratch_types=[
            pltpu.VMEM_SHARED((N, FQ), jnp.float32),
            pltpu.VMEM((SUP, CH), jnp.int32),
            pltpu.VMEM((SUP, CH), jnp.int32),
            pltpu.VMEM((SUP, CH), jnp.float32),
            pltpu.VMEM((SUP, CH, FQ), jnp.float32),
            pltpu.VMEM((1, CPY), jnp.int32),
            pltpu.VMEM((CPY, FQ), jnp.float32),
            pltpu.SemaphoreType.DMA,
            pltpu.SemaphoreType.DMA,
        ],
        compiler_params=_sc_params(),
    )
    def mp_kernel(h_hbm, row_hbm, col_hbm, w_hbm, z_hbm, out_hbm,
                  acc_s, ri, ci, wv, gb, oi, cb, gsem, ssem):
        c = lax.axis_index("c")
        s = lax.axis_index("s")
        off = s * PART
        iota4 = lax.iota(jnp.int32, 16) * 4

        for p in range(2):
            q = 2 * p + c

            @pl.when(s < 15)
            def _():
                pltpu.sync_copy(z_hbm, acc_s.at[pl.ds(off, PART)])

            @pl.when(s == 15)
            def _():
                pltpu.sync_copy(z_hbm.at[pl.ds(0, LAST)],
                                acc_s.at[pl.ds(off, LAST)])

            plsc.subcore_barrier()

            @pl.loop(0, rows_s // SUP)
            def _(u):
                base = s * rows_s + u * SUP
                pltpu.sync_copy(row_hbm.at[pl.ds(base, SUP)], ri)
                pltpu.sync_copy(col_hbm.at[pl.ds(base, SUP)], ci)
                pltpu.sync_copy(w_hbm.at[pl.ds(base, SUP)], wv)
                for b in range(SUP):
                    for k in range(CH // 16):
                        sl = pl.ds(k * 16, 16)
                        ri[b, sl] = ri[b, sl] * 4 + q
                gds = [pltpu.async_copy(h_hbm.at[ri.at[b]], gb.at[b], gsem)
                       for b in range(SUP)]
                for d in gds:
                    d.wait()

                for b in range(SUP):
                    for g in range(CH // 16):
                        w16 = wv[b, pl.ds(g * 16, 16)]
                        for j in range(16):
                            e = g * 16 + j
                            w = _lane_splat(w16, j)
                            gb[b, e, pl.ds(0, FQ)] = (
                                gb[b, e, pl.ds(0, FQ)] * w)

                sds = [pltpu.async_copy(gb.at[b], acc_s.at[ci.at[b]], ssem,
                                        add=True) for b in range(SUP)]
                for d in sds:
                    d.wait()

            plsc.subcore_barrier()

            @pl.loop(0, (NCPY + NSUB - 1) // NSUB)
            def _(k):
                t = s + k * NSUB

                @pl.when(t < NCPY)
                def _():
                    for g in range(CPY // 16):
                        oi[0, pl.ds(g * 16, 16)] = (
                            iota4 + ((t * CPY + g * 16) * 4 + q))
                    pltpu.sync_copy(acc_s.at[pl.ds(t * CPY, CPY)], cb)
                    pltpu.sync_copy(cb, out_hbm.at[oi.at[0]])

            if p == 0:
                plsc.subcore_barrier()

    return mp_kernel(h4, row2, col2, w2, z16)


def _mm1_body(x_ref, w_ref, b_ref, o_ref):
    o_ref[...] = jnp.dot(x_ref[...], w_ref[...]) + b_ref[...]


def _tc_mm1(x, W1, b1):
    return pl.pallas_call(
        _mm1_body,
        grid=(N // BLK,),
        in_specs=[
            pl.BlockSpec((BLK, F), lambda i: (i, 0)),
            pl.BlockSpec((F, F), lambda i: (0, 0)),
            pl.BlockSpec((1, F), lambda i: (0, 0)),
        ],
        out_specs=pl.BlockSpec((BLK, F), lambda i: (i, 0)),
        out_shape=jax.ShapeDtypeStruct((N, F), jnp.float32),
    )(x, W1, b1.reshape(1, F))


def _ds_body(degp_ref, t_ref, ds_ref, hs_ref):
    i = pl.program_id(0)
    deg = degp_ref[0][:, 0:1] + degp_ref[1][:, 0:1] + 1.0
    rows = lax.broadcasted_iota(jnp.int32, deg.shape, 0)
    deg = jnp.where((rows == 0) & (i == 0), deg - float(PAD), deg)
    ds = jnp.broadcast_to(lax.rsqrt(deg), t_ref.shape)
    ds_ref[...] = ds
    hs_ref[...] = ds * t_ref[...]


def _tc_ds_hs(degp, t1):
    blk = 3216
    return pl.pallas_call(
        _ds_body,
        grid=(N // blk,),
        in_specs=[
            pl.BlockSpec((2, blk, 16), lambda i: (0, i, 0)),
            pl.BlockSpec((blk, F), lambda i: (i, 0)),
        ],
        out_specs=[
            pl.BlockSpec((blk, F), lambda i: (i, 0)),
            pl.BlockSpec((blk, F), lambda i: (i, 0)),
        ],
        out_shape=[
            jax.ShapeDtypeStruct((N, F), jnp.float32),
            jax.ShapeDtypeStruct((N, F), jnp.float32),
        ],
    )(degp, t1)


def _layer_body(acc_ref, hs_ref, ds_ref, lw_ref, w_ref, b_ref, o_ref):
    ds = ds_ref[...]
    act = jax.nn.relu(ds * (acc_ref[...] + lw_ref[...] * hs_ref[...]))
    o_ref[...] = ds * (jnp.dot(act, w_ref[...]) + b_ref[...])


def _tc_layer(acc, hs, ds, lw, W, b):
    return pl.pallas_call(
        _layer_body,
        grid=(N // BLK,),
        in_specs=[
            pl.BlockSpec((BLK, F), lambda i: (i, 0)),
            pl.BlockSpec((BLK, F), lambda i: (i, 0)),
            pl.BlockSpec((BLK, F), lambda i: (i, 0)),
            pl.BlockSpec((BLK, F), lambda i: (i, 0)),
            pl.BlockSpec((F, F), lambda i: (0, 0)),
            pl.BlockSpec((1, F), lambda i: (0, 0)),
        ],
        out_specs=pl.BlockSpec((BLK, F), lambda i: (i, 0)),
        out_shape=jax.ShapeDtypeStruct((N, F), jnp.float32),
    )(acc, hs, ds, lw, W, b.reshape(1, F))


def _final_body(acc_ref, hs_ref, ds_ref, lw_ref, o_ref):
    o_ref[...] = jax.nn.relu(
        ds_ref[...] * (acc_ref[...] + lw_ref[...] * hs_ref[...]))


def _tc_final(acc, hs, ds, lw):
    return pl.pallas_call(
        _final_body,
        grid=(N // BLK,),
        in_specs=[
            pl.BlockSpec((BLK, F), lambda i: (i, 0)),
            pl.BlockSpec((BLK, F), lambda i: (i, 0)),
            pl.BlockSpec((BLK, F), lambda i: (i, 0)),
            pl.BlockSpec((BLK, F), lambda i: (i, 0)),
        ],
        out_specs=pl.BlockSpec((BLK, F), lambda i: (i, 0)),
        out_shape=jax.ShapeDtypeStruct((N, F), jnp.float32),
    )(acc, hs, ds, lw)


def _decoder_body(h_ref, ai_ref, bi_ref, p1_ref, p2_ref, o_ref, a_scr, b_scr):
    def gather(i, _):
        a_scr[pl.ds(i, 1)] = h_ref[pl.ds(ai_ref[i], 1)]
        b_scr[pl.ds(i, 1)] = h_ref[pl.ds(bi_ref[i], 1)]
        return 0

    lax.fori_loop(0, 512, gather, 0)
    p1 = p1_ref[...]
    t = jnp.dot(jnp.dot(jnp.dot(a_scr[...], p1), p2_ref[...]), p1.T)
    o_ref[...] = jnp.sum(t * b_scr[...], axis=1, keepdims=True)


def _tc_decoder(h3, ai, bi, P1, P2):
    return pl.pallas_call(
        _decoder_body,
        in_specs=[
            pl.BlockSpec(memory_space=pltpu.VMEM),
            pl.BlockSpec(memory_space=pltpu.SMEM),
            pl.BlockSpec(memory_space=pltpu.SMEM),
            pl.BlockSpec(memory_space=pltpu.VMEM),
            pl.BlockSpec(memory_space=pltpu.VMEM),
        ],
        out_specs=pl.BlockSpec(memory_space=pltpu.VMEM),
        out_shape=jax.ShapeDtypeStruct((512, 1), jnp.float32),
        scratch_shapes=[
            pltpu.VMEM((512, F), jnp.float32),
            pltpu.VMEM((512, F), jnp.float32),
        ],
    )(h3, ai, bi, P1, P2)


def kernel(x, edge_index, drug_index, label, W1, b1, ge1, lge1, W2, b2, ge2,
           lge2, W3, b3, ge3, lge3, P1, P2):
    del label
    i32 = jnp.int32
    f32 = jnp.float32

    row = edge_index[0].astype(i32)
    col = edge_index[1].astype(i32)
    zpad = jnp.zeros((PAD,), i32)
    row2 = jnp.concatenate([row, zpad]).reshape(E_PAD // CH, CH)
    col2 = jnp.concatenate([col, zpad]).reshape(E_PAD // CH, CH)

    ones_drug = jnp.ones((NUM_DRUG_EDGE,), f32)
    wpad = jnp.zeros((PAD,), f32)

    def edge_w(ge):
        w = jnp.concatenate(
            [jnp.tile(jnp.concatenate([ge, ones_drug]), GRAPH_BATCH), wpad])
        return w.reshape(E_PAD // CH, CH)

    ones_loop = jnp.ones((NUM_DRUG,), f32)

    def loop_w(lge):
        lw = jnp.tile(jnp.concatenate([lge, ones_loop]), GRAPH_BATCH)
        return jnp.broadcast_to(lw[:, None], (N, F))

    z16 = jnp.zeros((PART, 16), f32)
    ones16 = jnp.ones((CH, 16), f32)

    degp = _sc_degree(col2, z16, ones16)
    t1 = _tc_mm1(x, W1, b1)
    ds, hs = _tc_ds_hs(degp, t1)

    acc = _sc_message_pass(hs.reshape(4 * N, FQ), row2, col2,
                           edge_w(ge1), z16).reshape(N, F)
    hs = _tc_layer(acc, hs, ds, loop_w(lge1), W2, b2)
    acc = _sc_message_pass(hs.reshape(4 * N, FQ), row2, col2,
                           edge_w(ge2), z16).reshape(N, F)
    hs = _tc_layer(acc, hs, ds, loop_w(lge2), W3, b3)
    acc = _sc_message_pass(hs.reshape(4 * N, FQ), row2, col2,
                           edge_w(ge3), z16).reshape(N, F)
    h3 = _tc_final(acc, hs, ds, loop_w(lge3))

    idx = drug_index.reshape(-1, 2).astype(i32)
    ai = (idx[:, 0] - 1) % N
    bi = (idx[:, 1] - 1) % N
    return _tc_decoder(h3, ai, bi, P1, P2)


# speedup vs baseline: 15.2486x; 1.3735x over previous
import dataclasses
import functools

import jax
import jax.numpy as jnp
from jax import lax
from jax.experimental import pallas as pl
from jax.experimental.pallas import tpu as pltpu
from jax.experimental.pallas import tpu_sc as plsc

NODE_NUM = 8040
GRAPH_BATCH = 8
N = NODE_NUM * GRAPH_BATCH
E = 125000 * GRAPH_BATCH
NUM_DRUG_EDGE = 25000
NUM_DRUG = 38
F = 64
FQ = 16

NCORE = 2
NSUB = 16
CH = 128
SUP = 7

EPS = 62720
E_PAD = NSUB * EPS
PAD = E_PAD - E
EPW_DEG = E_PAD // (NCORE * NSUB)

PART = 4024
LAST = N - 15 * PART

CPY = 96
NCPY = N // CPY

BLK = 6432
HIGH = lax.Precision.HIGHEST


@functools.lru_cache(maxsize=None)
def _sc_params():
    cp = pltpu.CompilerParams()
    if "needs_layout_passes" in pltpu.CompilerParams.__dataclass_fields__:
        cp = dataclasses.replace(cp, needs_layout_passes=False)
    if "use_tc_tiling_on_sc" in pltpu.CompilerParams.__dataclass_fields__:
        cp = dataclasses.replace(cp, use_tc_tiling_on_sc=False)
    return cp


@functools.lru_cache(maxsize=None)
def _mesh():
    return plsc.VectorSubcoreMesh(core_axis_name="c", subcore_axis_name="s",
                                  num_cores=NCORE, num_subcores=NSUB)


def _sc_degree(col2, z16, ones16):
    rows_w = EPW_DEG // CH

    @functools.partial(
        pl.kernel,
        out_type=jax.ShapeDtypeStruct((NCORE, N, 16), jnp.float32),
        mesh=_mesh(),
        scratch_types=[
            pltpu.VMEM_SHARED((N, 16), jnp.float32),
            pltpu.VMEM((SUP, CH), jnp.int32),
            pltpu.VMEM((CH, 16), jnp.float32),
            pltpu.SemaphoreType.DMA,
        ],
        compiler_params=_sc_params(),
    )
    def deg_kernel(col_hbm, z_hbm, ones_hbm, out_hbm, acc_s, ci, ones_v, sem):
        c = lax.axis_index("c")
        s = lax.axis_index("s")
        wid = s * NCORE + c
        off = s * PART

        @pl.when(s < 15)
        def _():
            pltpu.sync_copy(z_hbm, acc_s.at[pl.ds(off, PART)])

        @pl.when(s == 15)
        def _():
            pltpu.sync_copy(z_hbm.at[pl.ds(0, LAST)],
                            acc_s.at[pl.ds(off, LAST)])

        pltpu.sync_copy(ones_hbm, ones_v)
        plsc.subcore_barrier()

        @pl.loop(0, rows_w // SUP)
        def _(u):
            base = wid * rows_w + u * SUP
            pltpu.sync_copy(col_hbm.at[pl.ds(base, SUP)], ci)
            ds_ = [pltpu.async_copy(ones_v, acc_s.at[ci.at[b]], sem,
                                    add=True) for b in range(SUP)]
            for d in ds_:
                d.wait()

        plsc.subcore_barrier()

        @pl.when(s < 15)
        def _():
            pltpu.sync_copy(acc_s.at[pl.ds(off, PART)],
                            out_hbm.at[c, pl.ds(off, PART)])

        @pl.when(s == 15)
        def _():
            pltpu.sync_copy(acc_s.at[pl.ds(off, LAST)],
                            out_hbm.at[c, pl.ds(off, LAST)])

    return deg_kernel(col2, z16, ones16)


_GATHER_DNUMS = lax.GatherDimensionNumbers(
    offset_dims=(), collapsed_slice_dims=(0,), start_index_map=(0,))


def _lane_splat(vec16, j):
    idx = jnp.full((16, 1), j, jnp.int32)
    return lax.gather(vec16, idx, _GATHER_DNUMS, (1,),
                      mode=lax.GatherScatterMode.PROMISE_IN_BOUNDS)


def _sc_message_pass(h4, row2, col2, w2, z16):
    rows_s = EPS // CH

    nsup = rows_s // SUP

    @functools.partial(
        pl.kernel,
        out_type=jax.ShapeDtypeStruct((4 * N, FQ), jnp.float32),
        mesh=_mesh(),
        scratch_types=[
            pltpu.VMEM_SHARED((N, FQ), jnp.float32),
            pltpu.VMEM((2 * SUP, CH), jnp.int32),
            pltpu.VMEM((2 * SUP, CH), jnp.int32),
            pltpu.VMEM((2 * SUP, CH), jnp.float32),
            pltpu.VMEM((2 * SUP, CH, FQ), jnp.float32),
            pltpu.VMEM((2, CPY), jnp.int32),
            pltpu.VMEM((2, CPY, FQ), jnp.float32),
            pltpu.SemaphoreType.DMA,
            pltpu.SemaphoreType.DMA,
            pltpu.SemaphoreType.DMA,
            pltpu.SemaphoreType.DMA,
        ],
        compiler_params=_sc_params(),
    )
    def mp_kernel(h_hbm, row_hbm, col_hbm, w_hbm, z_hbm, out_hbm,
                  acc_s, ri, ci, wv, gb, oi, cb, isem, gsem, ssem, osem):
        c = lax.axis_index("c")
        s = lax.axis_index("s")
        off = s * PART
        iota4 = lax.iota(jnp.int32, 16) * 4

        def fire_idx(u, sl):
            base = s * rows_s + u * SUP
            sll = pl.ds(sl * SUP, SUP)
            pltpu.async_copy(row_hbm.at[pl.ds(base, SUP)], ri.at[sll], isem)
            pltpu.async_copy(col_hbm.at[pl.ds(base, SUP)], ci.at[sll], isem)
            pltpu.async_copy(w_hbm.at[pl.ds(base, SUP)], wv.at[sll], isem)

        def drain_idx(sl):
            src = row_hbm.at[pl.ds(0, SUP)]
            sll = pl.ds(sl * SUP, SUP)
            pltpu.make_async_copy(src, ri.at[sll], isem).wait()
            pltpu.make_async_copy(src, ci.at[sll], isem).wait()
            wsrc = w_hbm.at[pl.ds(0, SUP)]
            pltpu.make_async_copy(wsrc, wv.at[sll], isem).wait()

        def transform(sl, q):
            for b in range(SUP):
                j = sl * SUP + b
                for k in range(CH // 16):
                    slc = pl.ds(k * 16, 16)
                    ri[j, slc] = ri[j, slc] * 4 + q

        def fire_gather(sl):
            for b in range(SUP):
                j = sl * SUP + b
                pltpu.async_copy(h_hbm.at[ri.at[j]], gb.at[j], gsem)

        def drain_gather(sl):
            for b in range(SUP):
                j = sl * SUP + b
                pltpu.make_async_copy(h_hbm.at[pl.ds(0, CH)], gb.at[j],
                                      gsem).wait()

        def multiply(sl):
            @pl.loop(0, SUP)
            def _(b):
                j = sl * SUP + b
                for g in range(CH // 16):
                    w16 = wv[j, pl.ds(g * 16, 16)]
                    for jj in range(16):
                        e = g * 16 + jj
                        w = _lane_splat(w16, jj)
                        gb[j, e, pl.ds(0, FQ)] = gb[j, e, pl.ds(0, FQ)] * w

        def fire_scatter(sl):
            for b in range(SUP):
                j = sl * SUP + b
                pltpu.async_copy(gb.at[j], acc_s.at[ci.at[j]], ssem, add=True)

        def drain_scatter(sl):
            for b in range(SUP):
                j = sl * SUP + b
                pltpu.make_async_copy(h_hbm.at[pl.ds(0, CH)], gb.at[j],
                                      ssem).wait()

        def mid(u, sl_prev, sl_new, q, first):
            if not first:
                drain_scatter(sl_new)
            fire_idx(u, sl_new)
            drain_gather(sl_prev)
            multiply(sl_prev)
            fire_scatter(sl_prev)
            drain_idx(sl_new)
            transform(sl_new, q)
            fire_gather(sl_new)

        for p in range(2):
            q = 2 * p + c

            @pl.when(s < 15)
            def _():
                pltpu.sync_copy(z_hbm, acc_s.at[pl.ds(off, PART)])

            @pl.when(s == 15)
            def _():
                pltpu.sync_copy(z_hbm.at[pl.ds(0, LAST)],
                                acc_s.at[pl.ds(off, LAST)])

            plsc.subcore_barrier()

            fire_idx(0, 0)
            drain_idx(0)
            transform(0, q)
            fire_gather(0)
            mid(1, 0, 1, q, first=True)

            @pl.loop(0, (nsup - 2) // 2)
            def _(k):
                u = 2 * k + 2
                mid(u, 1, 0, q, first=False)
                mid(u + 1, 0, 1, q, first=False)

            drain_scatter(0)
            drain_gather(1)
            multiply(1)
            fire_scatter(1)
            drain_scatter(1)

            plsc.subcore_barrier()

            def cp_chunk(k, slot, drain):
                t = s * 42 + k
                if drain:
                    pltpu.make_async_copy(h_hbm.at[pl.ds(0, CPY)],
                                          cb.at[slot], osem).wait()
                for g in range(CPY // 16):
                    oi[slot, pl.ds(g * 16, 16)] = (
                        iota4 + ((t * CPY + g * 16) * 4 + q))
                pltpu.sync_copy(acc_s.at[pl.ds(t * CPY, CPY)], cb.at[slot])
                pltpu.async_copy(cb.at[slot], out_hbm.at[oi.at[slot]], osem)

            def cp_tail():
                pltpu.make_async_copy(h_hbm.at[pl.ds(0, CPY)], cb.at[0],
                                      osem).wait()
                pltpu.make_async_copy(h_hbm.at[pl.ds(0, CPY)], cb.at[1],
                                      osem).wait()

            @pl.when(s < 15)
            def _():
                cp_chunk(0, 0, drain=False)
                cp_chunk(1, 1, drain=False)

                @pl.loop(0, 20)
                def _(m):
                    cp_chunk(2 * m + 2, 0, drain=True)
                    cp_chunk(2 * m + 3, 1, drain=True)

                cp_tail()

            @pl.when(s == 15)
            def _():
                cp_chunk(0, 0, drain=False)
                cp_chunk(1, 1, drain=False)

                @pl.loop(0, 19)
                def _(m):
                    cp_chunk(2 * m + 2, 0, drain=True)
                    cp_chunk(2 * m + 3, 1, drain=True)

                cp_tail()

            if p == 0:
                plsc.subcore_barrier()

    return mp_kernel(h4, row2, col2, w2, z16)


def _mm1_body(x_ref, w_ref, b_ref, o_ref):
    o_ref[...] = jnp.dot(x_ref[...], w_ref[...]) + b_ref[...]


def _tc_mm1(x, W1, b1):
    return pl.pallas_call(
        _mm1_body,
        grid=(N // BLK,),
        in_specs=[
            pl.BlockSpec((BLK, F), lambda i: (i, 0)),
            pl.BlockSpec((F, F), lambda i: (0, 0)),
            pl.BlockSpec((1, F), lambda i: (0, 0)),
        ],
        out_specs=pl.BlockSpec((BLK, F), lambda i: (i, 0)),
        out_shape=jax.ShapeDtypeStruct((N, F), jnp.float32),
    )(x, W1, b1.reshape(1, F))


def _ds_body(degp_ref, t_ref, ds_ref, hs_ref):
    i = pl.program_id(0)
    deg = degp_ref[0][:, 0:1] + degp_ref[1][:, 0:1] + 1.0
    rows = lax.broadcasted_iota(jnp.int32, deg.shape, 0)
    deg = jnp.where((rows == 0) & (i == 0), deg - float(PAD), deg)
    ds = jnp.broadcast_to(lax.rsqrt(deg), t_ref.shape)
    ds_ref[...] = ds
    hs_ref[...] = ds * t_ref[...]


def _tc_ds_hs(degp, t1):
    blk = 3216
    return pl.pallas_call(
        _ds_body,
        grid=(N // blk,),
        in_specs=[
            pl.BlockSpec((2, blk, 16), lambda i: (0, i, 0)),
            pl.BlockSpec((blk, F), lambda i: (i, 0)),
        ],
        out_specs=[
            pl.BlockSpec((blk, F), lambda i: (i, 0)),
            pl.BlockSpec((blk, F), lambda i: (i, 0)),
        ],
        out_shape=[
            jax.ShapeDtypeStruct((N, F), jnp.float32),
            jax.ShapeDtypeStruct((N, F), jnp.float32),
        ],
    )(degp, t1)


def _layer_body(acc_ref, hs_ref, ds_ref, lw_ref, w_ref, b_ref, o_ref):
    ds = ds_ref[...]
    act = jax.nn.relu(ds * (acc_ref[...] + lw_ref[...] * hs_ref[...]))
    o_ref[...] = ds * (jnp.dot(act, w_ref[...]) + b_ref[...])


def _tc_layer(acc, hs, ds, lw, W, b):
    return pl.pallas_call(
        _layer_body,
        grid=(N // BLK,),
        in_specs=[
            pl.BlockSpec((BLK, F), lambda i: (i, 0)),
            pl.BlockSpec((BLK, F), lambda i: (i, 0)),
            pl.BlockSpec((BLK, F), lambda i: (i, 0)),
            pl.BlockSpec((BLK, F), lambda i: (i, 0)),
            pl.BlockSpec((F, F), lambda i: (0, 0)),
            pl.BlockSpec((1, F), lambda i: (0, 0)),
        ],
        out_specs=pl.BlockSpec((BLK, F), lambda i: (i, 0)),
        out_shape=jax.ShapeDtypeStruct((N, F), jnp.float32),
    )(acc, hs, ds, lw, W, b.reshape(1, F))


def _final_body(acc_ref, hs_ref, ds_ref, lw_ref, o_ref):
    o_ref[...] = jax.nn.relu(
        ds_ref[...] * (acc_ref[...] + lw_ref[...] * hs_ref[...]))


def _tc_final(acc, hs, ds, lw):
    return pl.pallas_call(
        _final_body,
        grid=(N // BLK,),
        in_specs=[
            pl.BlockSpec((BLK, F), lambda i: (i, 0)),
            pl.BlockSpec((BLK, F), lambda i: (i, 0)),
            pl.BlockSpec((BLK, F), lambda i: (i, 0)),
            pl.BlockSpec((BLK, F), lambda i: (i, 0)),
        ],
        out_specs=pl.BlockSpec((BLK, F), lambda i: (i, 0)),
        out_shape=jax.ShapeDtypeStruct((N, F), jnp.float32),
    )(acc, hs, ds, lw)


def _decoder_body(h_ref, ai_ref, bi_ref, p1_ref, p2_ref, o_ref, a_scr, b_scr):
    def gather(i, _):
        a_scr[pl.ds(i, 1)] = h_ref[pl.ds(ai_ref[i], 1)]
        b_scr[pl.ds(i, 1)] = h_ref[pl.ds(bi_ref[i], 1)]
        return 0

    lax.fori_loop(0, 512, gather, 0)
    p1 = p1_ref[...]
    t = jnp.dot(jnp.dot(jnp.dot(a_scr[...], p1), p2_ref[...]), p1.T)
    o_ref[...] = jnp.sum(t * b_scr[...], axis=1, keepdims=True)


def _tc_decoder(h3, ai, bi, P1, P2):
    return pl.pallas_call(
        _decoder_body,
        in_specs=[
            pl.BlockSpec(memory_space=pltpu.VMEM),
            pl.BlockSpec(memory_space=pltpu.SMEM),
            pl.BlockSpec(memory_space=pltpu.SMEM),
            pl.BlockSpec(memory_space=pltpu.VMEM),
            pl.BlockSpec(memory_space=pltpu.VMEM),
        ],
        out_specs=pl.BlockSpec(memory_space=pltpu.VMEM),
        out_shape=jax.ShapeDtypeStruct((512, 1), jnp.float32),
        scratch_shapes=[
            pltpu.VMEM((512, F), jnp.float32),
            pltpu.VMEM((512, F), jnp.float32),
        ],
    )(h3, ai, bi, P1, P2)


def kernel(x, edge_index, drug_index, label, W1, b1, ge1, lge1, W2, b2, ge2,
           lge2, W3, b3, ge3, lge3, P1, P2):
    del label
    i32 = jnp.int32
    f32 = jnp.float32

    row = edge_index[0].astype(i32)
    col = edge_index[1].astype(i32)
    zpad = jnp.zeros((PAD,), i32)
    row2 = jnp.concatenate([row, zpad]).reshape(E_PAD // CH, CH)
    col2 = jnp.concatenate([col, zpad]).reshape(E_PAD // CH, CH)

    ones_drug = jnp.ones((NUM_DRUG_EDGE,), f32)
    wpad = jnp.zeros((PAD,), f32)

    def edge_w(ge):
        w = jnp.concatenate(
            [jnp.tile(jnp.concatenate([ge, ones_drug]), GRAPH_BATCH), wpad])
        return w.reshape(E_PAD // CH, CH)

    ones_loop = jnp.ones((NUM_DRUG,), f32)

    def loop_w(lge):
        lw = jnp.tile(jnp.concatenate([lge, ones_loop]), GRAPH_BATCH)
        return jnp.broadcast_to(lw[:, None], (N, F))

    z16 = jnp.zeros((PART, 16), f32)
    ones16 = jnp.ones((CH, 16), f32)

    degp = _sc_degree(col2, z16, ones16)
    t1 = _tc_mm1(x, W1, b1)
    ds, hs = _tc_ds_hs(degp, t1)

    acc = _sc_message_pass(hs.reshape(4 * N, FQ), row2, col2,
                           edge_w(ge1), z16).reshape(N, F)
    hs = _tc_layer(acc, hs, ds, loop_w(lge1), W2, b2)
    acc = _sc_message_pass(hs.reshape(4 * N, FQ), row2, col2,
                           edge_w(ge2), z16).reshape(N, F)
    hs = _tc_layer(acc, hs, ds, loop_w(lge2), W3, b3)
    acc = _sc_message_pass(hs.reshape(4 * N, FQ), row2, col2,
                           edge_w(ge3), z16).reshape(N, F)
    h3 = _tc_final(acc, hs, ds, loop_w(lge3))

    idx = drug_index.reshape(-1, 2).astype(i32)
    ai = (idx[:, 0] - 1) % N
    bi = (idx[:, 1] - 1) % N
    return _tc_decoder(h3, ai, bi, P1, P2)


# speedup vs baseline: 18.2518x; 1.1970x over previous
import dataclasses
import functools

import jax
import jax.numpy as jnp
from jax import lax
from jax.experimental import pallas as pl
from jax.experimental.pallas import tpu as pltpu
from jax.experimental.pallas import tpu_sc as plsc

NODE_NUM = 8040
GRAPH_BATCH = 8
N = NODE_NUM * GRAPH_BATCH
E = 125000 * GRAPH_BATCH
NUM_DRUG_EDGE = 25000
NUM_DRUG = 38
F = 64
FQ = 16

NCORE = 2
NSUB = 16
CH = 128
SUP = 7

EPS = 62720
E_PAD = NSUB * EPS
PAD = E_PAD - E
EPW_DEG = E_PAD // (NCORE * NSUB)

PART = 4024
LAST = N - 15 * PART

CPY = 96
NCPY = N // CPY

BLK = 6432
HIGH = lax.Precision.HIGHEST


@functools.lru_cache(maxsize=None)
def _sc_params():
    cp = pltpu.CompilerParams()
    if "needs_layout_passes" in pltpu.CompilerParams.__dataclass_fields__:
        cp = dataclasses.replace(cp, needs_layout_passes=False)
    if "use_tc_tiling_on_sc" in pltpu.CompilerParams.__dataclass_fields__:
        cp = dataclasses.replace(cp, use_tc_tiling_on_sc=False)
    return cp


@functools.lru_cache(maxsize=None)
def _mesh():
    return plsc.VectorSubcoreMesh(core_axis_name="c", subcore_axis_name="s",
                                  num_cores=NCORE, num_subcores=NSUB)


def _sc_degree(col2, z16, ones16):
    rows_w = EPW_DEG // CH

    @functools.partial(
        pl.kernel,
        out_type=jax.ShapeDtypeStruct((NCORE, N, 16), jnp.float32),
        mesh=_mesh(),
        scratch_types=[
            pltpu.VMEM_SHARED((N, 16), jnp.float32),
            pltpu.VMEM((SUP, CH), jnp.int32),
            pltpu.VMEM((CH, 16), jnp.float32),
            pltpu.SemaphoreType.DMA,
        ],
        compiler_params=_sc_params(),
    )
    def deg_kernel(col_hbm, z_hbm, ones_hbm, out_hbm, acc_s, ci, ones_v, sem):
        c = lax.axis_index("c")
        s = lax.axis_index("s")
        wid = s * NCORE + c
        off = s * PART

        @pl.when(s < 15)
        def _():
            pltpu.sync_copy(z_hbm, acc_s.at[pl.ds(off, PART)])

        @pl.when(s == 15)
        def _():
            pltpu.sync_copy(z_hbm.at[pl.ds(0, LAST)],
                            acc_s.at[pl.ds(off, LAST)])

        pltpu.sync_copy(ones_hbm, ones_v)
        plsc.subcore_barrier()

        @pl.loop(0, rows_w // SUP)
        def _(u):
            base = wid * rows_w + u * SUP
            pltpu.sync_copy(col_hbm.at[pl.ds(base, SUP)], ci)
            ds_ = [pltpu.async_copy(ones_v, acc_s.at[ci.at[b]], sem,
                                    add=True) for b in range(SUP)]
            for d in ds_:
                d.wait()

        plsc.subcore_barrier()

        @pl.when(s < 15)
        def _():
            pltpu.sync_copy(acc_s.at[pl.ds(off, PART)],
                            out_hbm.at[c, pl.ds(off, PART)])

        @pl.when(s == 15)
        def _():
            pltpu.sync_copy(acc_s.at[pl.ds(off, LAST)],
                            out_hbm.at[c, pl.ds(off, LAST)])

    return deg_kernel(col2, z16, ones16)


_GATHER_DNUMS = lax.GatherDimensionNumbers(
    offset_dims=(), collapsed_slice_dims=(0,), start_index_map=(0,))


def _lane_splat(vec16, j):
    idx = jnp.full((16, 1), j, jnp.int32)
    return lax.gather(vec16, idx, _GATHER_DNUMS, (1,),
                      mode=lax.GatherScatterMode.PROMISE_IN_BOUNDS)


def _sc_message_pass(h4, row2, col2, w2, z16):
    rows_s = EPS // CH

    nsup = rows_s // SUP

    @functools.partial(
        pl.kernel,
        out_type=jax.ShapeDtypeStruct((4 * N, FQ), jnp.float32),
        mesh=_mesh(),
        scratch_types=[
            pltpu.VMEM_SHARED((N, FQ), jnp.float32),
            pltpu.VMEM((3 * SUP, CH), jnp.int32),
            pltpu.VMEM((3 * SUP, CH), jnp.int32),
            pltpu.VMEM((3 * SUP, CH), jnp.float32),
            pltpu.VMEM((3 * SUP, CH, FQ), jnp.float32),
            pltpu.VMEM((2, CPY), jnp.int32),
            pltpu.VMEM((2, CPY, FQ), jnp.float32),
            pltpu.SemaphoreType.DMA,
            pltpu.SemaphoreType.DMA,
            pltpu.SemaphoreType.DMA,
            pltpu.SemaphoreType.DMA,
        ],
        compiler_params=_sc_params(),
    )
    def mp_kernel(h_hbm, row_hbm, col_hbm, w_hbm, z_hbm, out_hbm,
                  acc_s, ri, ci, wv, gb, oi, cb, isem, gsem, ssem, osem):
        c = lax.axis_index("c")
        s = lax.axis_index("s")
        off = s * PART
        iota4 = lax.iota(jnp.int32, 16) * 4

        def fire_idx(u, sl):
            base = s * rows_s + u * SUP
            sll = pl.ds(sl * SUP, SUP)
            pltpu.async_copy(row_hbm.at[pl.ds(base, SUP)], ri.at[sll], isem)
            pltpu.async_copy(col_hbm.at[pl.ds(base, SUP)], ci.at[sll], isem)
            pltpu.async_copy(w_hbm.at[pl.ds(base, SUP)], wv.at[sll], isem)

        def drain_idx(sl):
            src = row_hbm.at[pl.ds(0, SUP)]
            sll = pl.ds(sl * SUP, SUP)
            pltpu.make_async_copy(src, ri.at[sll], isem).wait()
            pltpu.make_async_copy(src, ci.at[sll], isem).wait()
            wsrc = w_hbm.at[pl.ds(0, SUP)]
            pltpu.make_async_copy(wsrc, wv.at[sll], isem).wait()

        def transform(sl, q):
            for b in range(SUP):
                j = sl * SUP + b
                for k in range(CH // 16):
                    slc = pl.ds(k * 16, 16)
                    ri[j, slc] = ri[j, slc] * 4 + q

        def fire_gather(sl):
            for b in range(SUP):
                j = sl * SUP + b
                pltpu.async_copy(h_hbm.at[ri.at[j]], gb.at[j], gsem)

        def drain_gather(sl):
            for b in range(SUP):
                j = sl * SUP + b
                pltpu.make_async_copy(h_hbm.at[pl.ds(0, CH)], gb.at[j],
                                      gsem).wait()

        def multiply(sl):
            @pl.loop(0, SUP)
            def _(b):
                j = sl * SUP + b
                for g in range(CH // 16):
                    w16 = wv[j, pl.ds(g * 16, 16)]
                    for jj in range(16):
                        e = g * 16 + jj
                        w = _lane_splat(w16, jj)
                        gb[j, e, pl.ds(0, FQ)] = gb[j, e, pl.ds(0, FQ)] * w

        def fire_scatter(sl):
            for b in range(SUP):
                j = sl * SUP + b
                pltpu.async_copy(gb.at[j], acc_s.at[ci.at[j]], ssem, add=True)

        def drain_scatter(sl):
            for b in range(SUP):
                j = sl * SUP + b
                pltpu.make_async_copy(h_hbm.at[pl.ds(0, CH)], gb.at[j],
                                      ssem).wait()

        def start_super(u, sl, q):
            fire_idx(u, sl)
            drain_idx(sl)
            transform(sl, q)
            fire_gather(sl)

        def mid(u, sl_cons, sl_new, q, first):
            if not first:
                drain_scatter(sl_new)
            fire_idx(u, sl_new)
            drain_gather(sl_cons)
            multiply(sl_cons)
            fire_scatter(sl_cons)
            drain_idx(sl_new)
            transform(sl_new, q)
            fire_gather(sl_new)

        for p in range(2):
            q = 2 * p + c

            @pl.when(s < 15)
            def _():
                pltpu.sync_copy(z_hbm, acc_s.at[pl.ds(off, PART)])

            @pl.when(s == 15)
            def _():
                pltpu.sync_copy(z_hbm.at[pl.ds(0, LAST)],
                                acc_s.at[pl.ds(off, LAST)])

            plsc.subcore_barrier()

            start_super(0, 0, q)
            start_super(1, 1, q)
            mid(2, 0, 2, q, first=True)
            mid(3, 1, 0, q, first=False)

            @pl.loop(0, (nsup - 4) // 3)
            def _(k):
                u = 3 * k + 4
                mid(u, 2, 1, q, first=False)
                mid(u + 1, 0, 2, q, first=False)
                mid(u + 2, 1, 0, q, first=False)

            drain_gather(2)
            multiply(2)
            fire_scatter(2)
            drain_gather(0)
            multiply(0)
            fire_scatter(0)
            drain_scatter(1)
            drain_scatter(2)
            drain_scatter(0)

            plsc.subcore_barrier()

            def cp_chunk(k, slot, drain):
                t = s * 42 + k
                if drain:
                    pltpu.make_async_copy(h_hbm.at[pl.ds(0, CPY)],
                                          cb.at[slot], osem).wait()
                for g in range(CPY // 16):
                    oi[slot, pl.ds(g * 16, 16)] = (
                        iota4 + ((t * CPY + g * 16) * 4 + q))
                pltpu.sync_copy(acc_s.at[pl.ds(t * CPY, CPY)], cb.at[slot])
                pltpu.async_copy(cb.at[slot], out_hbm.at[oi.at[slot]], osem)

            def cp_tail():
                pltpu.make_async_copy(h_hbm.at[pl.ds(0, CPY)], cb.at[0],
                                      osem).wait()
                pltpu.make_async_copy(h_hbm.at[pl.ds(0, CPY)], cb.at[1],
                                      osem).wait()

            @pl.when(s < 15)
            def _():
                cp_chunk(0, 0, drain=False)
                cp_chunk(1, 1, drain=False)

                @pl.loop(0, 20)
                def _(m):
                    cp_chunk(2 * m + 2, 0, drain=True)
                    cp_chunk(2 * m + 3, 1, drain=True)

                cp_tail()

            @pl.when(s == 15)
            def _():
                cp_chunk(0, 0, drain=False)
                cp_chunk(1, 1, drain=False)

                @pl.loop(0, 19)
                def _(m):
                    cp_chunk(2 * m + 2, 0, drain=True)
                    cp_chunk(2 * m + 3, 1, drain=True)

                cp_tail()

            if p == 0:
                plsc.subcore_barrier()

    return mp_kernel(h4, row2, col2, w2, z16)


def _mm1_body(x_ref, w_ref, b_ref, o_ref):
    o_ref[...] = jnp.dot(x_ref[...], w_ref[...]) + b_ref[...]


def _tc_mm1(x, W1, b1):
    return pl.pallas_call(
        _mm1_body,
        grid=(N // BLK,),
        in_specs=[
            pl.BlockSpec((BLK, F), lambda i: (i, 0)),
            pl.BlockSpec((F, F), lambda i: (0, 0)),
            pl.BlockSpec((1, F), lambda i: (0, 0)),
        ],
        out_specs=pl.BlockSpec((BLK, F), lambda i: (i, 0)),
        out_shape=jax.ShapeDtypeStruct((N, F), jnp.float32),
    )(x, W1, b1.reshape(1, F))


def _ds_body(degp_ref, t_ref, ds_ref, hs_ref):
    i = pl.program_id(0)
    deg = degp_ref[0][:, 0:1] + degp_ref[1][:, 0:1] + 1.0
    rows = lax.broadcasted_iota(jnp.int32, deg.shape, 0)
    deg = jnp.where((rows == 0) & (i == 0), deg - float(PAD), deg)
    ds = jnp.broadcast_to(lax.rsqrt(deg), t_ref.shape)
    ds_ref[...] = ds
    hs_ref[...] = ds * t_ref[...]


def _tc_ds_hs(degp, t1):
    blk = 3216
    return pl.pallas_call(
        _ds_body,
        grid=(N // blk,),
        in_specs=[
            pl.BlockSpec((2, blk, 16), lambda i: (0, i, 0)),
            pl.BlockSpec((blk, F), lambda i: (i, 0)),
        ],
        out_specs=[
            pl.BlockSpec((blk, F), lambda i: (i, 0)),
            pl.BlockSpec((blk, F), lambda i: (i, 0)),
        ],
        out_shape=[
            jax.ShapeDtypeStruct((N, F), jnp.float32),
            jax.ShapeDtypeStruct((N, F), jnp.float32),
        ],
    )(degp, t1)


def _layer_body(acc_ref, hs_ref, ds_ref, lw_ref, w_ref, b_ref, o_ref):
    ds = ds_ref[...]
    act = jax.nn.relu(ds * (acc_ref[...] + lw_ref[...] * hs_ref[...]))
    o_ref[...] = ds * (jnp.dot(act, w_ref[...]) + b_ref[...])


def _tc_layer(acc, hs, ds, lw, W, b):
    return pl.pallas_call(
        _layer_body,
        grid=(N // BLK,),
        in_specs=[
            pl.BlockSpec((BLK, F), lambda i: (i, 0)),
            pl.BlockSpec((BLK, F), lambda i: (i, 0)),
            pl.BlockSpec((BLK, F), lambda i: (i, 0)),
            pl.BlockSpec((BLK, F), lambda i: (i, 0)),
            pl.BlockSpec((F, F), lambda i: (0, 0)),
            pl.BlockSpec((1, F), lambda i: (0, 0)),
        ],
        out_specs=pl.BlockSpec((BLK, F), lambda i: (i, 0)),
        out_shape=jax.ShapeDtypeStruct((N, F), jnp.float32),
    )(acc, hs, ds, lw, W, b.reshape(1, F))


def _final_body(acc_ref, hs_ref, ds_ref, lw_ref, o_ref):
    o_ref[...] = jax.nn.relu(
        ds_ref[...] * (acc_ref[...] + lw_ref[...] * hs_ref[...]))


def _tc_final(acc, hs, ds, lw):
    return pl.pallas_call(
        _final_body,
        grid=(N // BLK,),
        in_specs=[
            pl.BlockSpec((BLK, F), lambda i: (i, 0)),
            pl.BlockSpec((BLK, F), lambda i: (i, 0)),
            pl.BlockSpec((BLK, F), lambda i: (i, 0)),
            pl.BlockSpec((BLK, F), lambda i: (i, 0)),
        ],
        out_specs=pl.BlockSpec((BLK, F), lambda i: (i, 0)),
        out_shape=jax.ShapeDtypeStruct((N, F), jnp.float32),
    )(acc, hs, ds, lw)


def _decoder_body(h_ref, ai_ref, bi_ref, p1_ref, p2_ref, o_ref, a_scr, b_scr):
    def gather(i, _):
        a_scr[pl.ds(i, 1)] = h_ref[pl.ds(ai_ref[i], 1)]
        b_scr[pl.ds(i, 1)] = h_ref[pl.ds(bi_ref[i], 1)]
        return 0

    lax.fori_loop(0, 512, gather, 0)
    p1 = p1_ref[...]
    t = jnp.dot(jnp.dot(jnp.dot(a_scr[...], p1), p2_ref[...]), p1.T)
    o_ref[...] = jnp.sum(t * b_scr[...], axis=1, keepdims=True)


def _tc_decoder(h3, ai, bi, P1, P2):
    return pl.pallas_call(
        _decoder_body,
        in_specs=[
            pl.BlockSpec(memory_space=pltpu.VMEM),
            pl.BlockSpec(memory_space=pltpu.SMEM),
            pl.BlockSpec(memory_space=pltpu.SMEM),
            pl.BlockSpec(memory_space=pltpu.VMEM),
            pl.BlockSpec(memory_space=pltpu.VMEM),
        ],
        out_specs=pl.BlockSpec(memory_space=pltpu.VMEM),
        out_shape=jax.ShapeDtypeStruct((512, 1), jnp.float32),
        scratch_shapes=[
            pltpu.VMEM((512, F), jnp.float32),
            pltpu.VMEM((512, F), jnp.float32),
        ],
    )(h3, ai, bi, P1, P2)


def kernel(x, edge_index, drug_index, label, W1, b1, ge1, lge1, W2, b2, ge2,
           lge2, W3, b3, ge3, lge3, P1, P2):
    del label
    i32 = jnp.int32
    f32 = jnp.float32

    row = edge_index[0].astype(i32)
    col = edge_index[1].astype(i32)
    zpad = jnp.zeros((PAD,), i32)
    row2 = jnp.concatenate([row, zpad]).reshape(E_PAD // CH, CH)
    col2 = jnp.concatenate([col, zpad]).reshape(E_PAD // CH, CH)

    ones_drug = jnp.ones((NUM_DRUG_EDGE,), f32)
    wpad = jnp.zeros((PAD,), f32)

    def edge_w(ge):
        w = jnp.concatenate(
            [jnp.tile(jnp.concatenate([ge, ones_drug]), GRAPH_BATCH), wpad])
        return w.reshape(E_PAD // CH, CH)

    ones_loop = jnp.ones((NUM_DRUG,), f32)

    def loop_w(lge):
        lw = jnp.tile(jnp.concatenate([lge, ones_loop]), GRAPH_BATCH)
        return jnp.broadcast_to(lw[:, None], (N, F))

    z16 = jnp.zeros((PART, 16), f32)
    ones16 = jnp.ones((CH, 16), f32)

    degp = _sc_degree(col2, z16, ones16)
    t1 = _tc_mm1(x, W1, b1)
    ds, hs = _tc_ds_hs(degp, t1)

    acc = _sc_message_pass(hs.reshape(4 * N, FQ), row2, col2,
                           edge_w(ge1), z16).reshape(N, F)
    hs = _tc_layer(acc, hs, ds, loop_w(lge1), W2, b2)
    acc = _sc_message_pass(hs.reshape(4 * N, FQ), row2, col2,
                           edge_w(ge2), z16).reshape(N, F)
    hs = _tc_layer(acc, hs, ds, loop_w(lge2), W3, b3)
    acc = _sc_message_pass(hs.reshape(4 * N, FQ), row2, col2,
                           edge_w(ge3), z16).reshape(N, F)
    h3 = _tc_final(acc, hs, ds, loop_w(lge3))

    idx = drug_index.reshape(-1, 2).astype(i32)
    ai = (idx[:, 0] - 1) % N
    bi = (idx[:, 1] - 1) % N
    return _tc_decoder(h3, ai, bi, P1, P2)
